# baseline (speedup 1.0000x reference)
import functools

import jax
import jax.numpy as jnp
import numpy as np
from jax.experimental import pallas as pl
from jax.experimental.pallas import tpu as pltpu

_CB = 2560


def _mm(a, b):
    return jax.lax.dot_general(
        a.astype(jnp.bfloat16), b.astype(jnp.bfloat16),
        (((1,), (0,)), ((), ())),
        preferred_element_type=jnp.float32)


def _xw1_body(x_ref, w1_ref, z1_ref):
    z1_ref[...] = _mm(x_ref[...], w1_ref[...]).astype(jnp.bfloat16)


def _pass_b_body(adj_ref, z1_ref, b1_ref, w2_ref, b2_ref, z2_ref, part_ref,
                 *, rb, n, nchunk):
    k = pl.program_id(0)

    @pl.when(k == 0)
    def _():
        z2_ref[...] = jnp.zeros_like(z2_ref)

    pre = _mm(adj_ref[...], z1_ref[...]) + b1_ref[...]
    h = jnp.maximum(pre, 0.0).astype(jnp.bfloat16)
    z2_ref[pl.ds(k * rb, rb), :] = _mm(h, w2_ref[...]).astype(jnp.bfloat16)

    def chunk_step(jc, acc):
        c0 = pl.multiple_of(jc * _CB, _CB)
        return acc + _mm(adj_ref[:, pl.ds(c0, _CB)], z2_ref[pl.ds(c0, _CB), :])

    c_hi = jnp.minimum(((k + 1) * rb) // _CB, nchunk - 1)
    acc0 = jnp.zeros_like(part_ref) + b2_ref[...]
    part_ref[...] = jax.lax.fori_loop(0, c_hi, chunk_step, acc0)


def _pass_c_body(s_ref, adj_ref, part_ref, z2_ref, wc_ref, bc_ref,
                 lsm_ref, cls_ref, out_sc, m_sc, sum_sc,
                 *, rb, n, nsteps, nchunk):
    t = pl.program_id(0)
    k = s_ref[0, t]
    jc = s_ref[1, t]
    first = s_ref[2, t] == 1
    tail = jc == nchunk - 1

    @pl.when(t == 0)
    def _():
        m_sc[...] = jnp.full_like(m_sc, -1e30)
        sum_sc[...] = jnp.zeros_like(sum_sc)

    a = adj_ref[...].astype(jnp.bfloat16)
    cols = jax.lax.broadcasted_iota(jnp.int32, (1, _CB), 1)
    a = jnp.where(cols < n - jc * _CB, a, jnp.bfloat16(0))
    contrib = jax.lax.dot_general(
        a, z2_ref[pl.ds(jc * _CB, _CB), :],
        (((1,), (0,)), ((), ())), preferred_element_type=jnp.float32)

    @pl.when(first)
    def _():
        out_sc[pl.ds(k * rb, rb), :] = part_ref[...] + contrib

    @pl.when(jnp.logical_not(first))
    def _():
        out_sc[pl.ds(k * rb, rb), :] = out_sc[pl.ds(k * rb, rb), :] + contrib

    @pl.when(tail)
    def _():
        acc = out_sc[pl.ds(k * rb, rb), :]
        cls_ref[...] = _mm(acc, wc_ref[...]) + bc_ref[...]
        m_old = m_sc[...]
        m_new = jnp.maximum(m_old, jnp.max(acc, axis=0, keepdims=True))
        sum_sc[...] = (sum_sc[...] * jnp.exp(m_old - m_new)
                       + jnp.sum(jnp.exp(acc - m_new), axis=0, keepdims=True))
        m_sc[...] = m_new

    @pl.when(t == nsteps - 1)
    def _():
        lse = m_sc[...] + jnp.log(sum_sc[...])
        lsm_ref[...] = out_sc[...] - lse


def kernel(x, adj, W1, b1, W2, b2, Wc, bc):
    n, nfeat = x.shape
    nhid = W1.shape[1]
    ncls = Wc.shape[1]
    split = 4576

    rb = 400 if n % 400 == 0 else n
    ng = n // rb
    nchunk = -(-n // _CB)
    npad = nchunk * _CB

    z1 = pl.pallas_call(
        _xw1_body,
        out_shape=jax.ShapeDtypeStruct((n, nhid), jnp.bfloat16),
    )(x, W1)

    z2, part = pl.pallas_call(
        functools.partial(_pass_b_body, rb=rb, n=n, nchunk=nchunk),
        grid=(ng,),
        in_specs=[
            pl.BlockSpec((rb, n), lambda k: (k, 0)),
            pl.BlockSpec((n, nhid), lambda k: (0, 0)),
            pl.BlockSpec((1, nhid), lambda k: (0, 0)),
            pl.BlockSpec((nhid, nfeat), lambda k: (0, 0)),
            pl.BlockSpec((1, nfeat), lambda k: (0, 0)),
        ],
        out_specs=[
            pl.BlockSpec((npad, nfeat), lambda k: (0, 0)),
            pl.BlockSpec((rb, nfeat), lambda k: (k, 0)),
        ],
        out_shape=[
            jax.ShapeDtypeStruct((npad, nfeat), jnp.bfloat16),
            jax.ShapeDtypeStruct((n, nfeat), jnp.float32),
        ],
    )(adj, z1, b1.reshape(1, -1), W2, b2.reshape(1, -1))

    ks, js, fs = [], [], []
    for kk in range(ng):
        c_hi = ((kk + 1) * rb) // _CB
        for jj in range(min(c_hi, nchunk - 1), nchunk):
            ks.append(kk)
            js.append(jj)
            fs.append(1 if jj == min(c_hi, nchunk - 1) else 0)
    sarr = jnp.asarray(np.array([ks, js, fs], dtype=np.int32))
    nsteps = len(ks)

    grid_spec = pltpu.PrefetchScalarGridSpec(
        num_scalar_prefetch=1,
        grid=(nsteps,),
        in_specs=[
            pl.BlockSpec((rb, _CB), lambda t, s: (s[0, t], s[1, t])),
            pl.BlockSpec((rb, nfeat), lambda t, s: (s[0, t], 0)),
            pl.BlockSpec((npad, nfeat), lambda t, s: (0, 0)),
            pl.BlockSpec((nfeat, ncls), lambda t, s: (0, 0)),
            pl.BlockSpec((1, ncls), lambda t, s: (0, 0)),
        ],
        out_specs=[
            pl.BlockSpec((n, nfeat), lambda t, s: (0, 0)),
            pl.BlockSpec((rb, ncls), lambda t, s: (s[0, t], 0)),
        ],
        scratch_shapes=[
            pltpu.VMEM((n, nfeat), jnp.float32),
            pltpu.VMEM((1, nfeat), jnp.float32),
            pltpu.VMEM((1, nfeat), jnp.float32),
        ],
    )
    lsm, cls = pl.pallas_call(
        functools.partial(_pass_c_body, rb=rb, n=n, nsteps=nsteps,
                          nchunk=nchunk),
        grid_spec=grid_spec,
        out_shape=[
            jax.ShapeDtypeStruct((n, nfeat), jnp.float32),
            jax.ShapeDtypeStruct((n, ncls), jnp.float32),
        ],
    )(sarr, adj, part, z2, Wc, bc.reshape(1, -1))

    return (lsm, cls[:split], cls[split:])


# speedup vs baseline: 1.0069x; 1.0069x over previous
import functools

import jax
import jax.numpy as jnp
import numpy as np
from jax.experimental import pallas as pl
from jax.experimental.pallas import tpu as pltpu

_CB = 2560


def _mm(a, b):
    return jax.lax.dot_general(
        a.astype(jnp.bfloat16), b.astype(jnp.bfloat16),
        (((1,), (0,)), ((), ())),
        preferred_element_type=jnp.float32)


def _xw1_body(x_ref, w1_ref, z1_ref):
    z1_ref[...] = _mm(x_ref[...], w1_ref[...]).astype(jnp.bfloat16)


def _pass_b_body(adj_ref, z1_ref, b1_ref, w2_ref, b2_ref, z2_ref, part_ref,
                 *, rb, n, nchunk):
    k = pl.program_id(0)

    @pl.when(k == 0)
    def _():
        z2_ref[...] = jnp.zeros_like(z2_ref)

    a = adj_ref[...].astype(jnp.bfloat16)
    pre = jax.lax.dot_general(
        a, z1_ref[...], (((1,), (0,)), ((), ())),
        preferred_element_type=jnp.float32) + b1_ref[...]
    h = jnp.maximum(pre, 0.0).astype(jnp.bfloat16)
    z2_ref[pl.ds(k * rb, rb), :] = _mm(h, w2_ref[...]).astype(jnp.bfloat16)

    c_hi = jnp.minimum(((k + 1) * rb) // _CB, nchunk - 1)
    part_ref[...] = jnp.zeros_like(part_ref) + b2_ref[...]
    for jc in range(nchunk - 1):
        @pl.when(jc < c_hi)
        def _(jc=jc):
            part_ref[...] = part_ref[...] + jax.lax.dot_general(
                a[:, jc * _CB:(jc + 1) * _CB],
                z2_ref[pl.ds(jc * _CB, _CB), :],
                (((1,), (0,)), ((), ())),
                preferred_element_type=jnp.float32)


def _pass_c_body(s_ref, adj_ref, part_ref, z2_ref, wc_ref, bc_ref,
                 lsm_ref, cls_ref, out_sc, m_sc, sum_sc,
                 *, rb, n, nsteps, nchunk):
    t = pl.program_id(0)
    k = s_ref[0, t]
    jc = s_ref[1, t]
    first = s_ref[2, t] == 1
    tail = jc == nchunk - 1

    @pl.when(t == 0)
    def _():
        m_sc[...] = jnp.full_like(m_sc, -1e30)
        sum_sc[...] = jnp.zeros_like(sum_sc)

    a = adj_ref[...].astype(jnp.bfloat16)
    cols = jax.lax.broadcasted_iota(jnp.int32, (1, _CB), 1)
    a = jnp.where(cols < n - jc * _CB, a, jnp.bfloat16(0))
    contrib = jax.lax.dot_general(
        a, z2_ref[pl.ds(jc * _CB, _CB), :],
        (((1,), (0,)), ((), ())), preferred_element_type=jnp.float32)

    @pl.when(first)
    def _():
        out_sc[pl.ds(k * rb, rb), :] = part_ref[...] + contrib

    @pl.when(jnp.logical_not(first))
    def _():
        out_sc[pl.ds(k * rb, rb), :] = out_sc[pl.ds(k * rb, rb), :] + contrib

    @pl.when(tail)
    def _():
        acc = out_sc[pl.ds(k * rb, rb), :]
        cls_ref[...] = _mm(acc, wc_ref[...]) + bc_ref[...]
        m_old = m_sc[...]
        m_new = jnp.maximum(m_old, jnp.max(acc, axis=0, keepdims=True))
        sum_sc[...] = (sum_sc[...] * jnp.exp(m_old - m_new)
                       + jnp.sum(jnp.exp(acc - m_new), axis=0, keepdims=True))
        m_sc[...] = m_new

    @pl.when(t == nsteps - 1)
    def _():
        lse = m_sc[...] + jnp.log(sum_sc[...])
        lsm_ref[...] = out_sc[...] - lse


def kernel(x, adj, W1, b1, W2, b2, Wc, bc):
    n, nfeat = x.shape
    nhid = W1.shape[1]
    ncls = Wc.shape[1]
    split = 4576

    rb = 400 if n % 400 == 0 else n
    ng = n // rb
    nchunk = -(-n // _CB)
    npad = nchunk * _CB

    z1 = pl.pallas_call(
        _xw1_body,
        out_shape=jax.ShapeDtypeStruct((n, nhid), jnp.bfloat16),
    )(x, W1)

    z2, part = pl.pallas_call(
        functools.partial(_pass_b_body, rb=rb, n=n, nchunk=nchunk),
        grid=(ng,),
        in_specs=[
            pl.BlockSpec((rb, n), lambda k: (k, 0)),
            pl.BlockSpec((n, nhid), lambda k: (0, 0)),
            pl.BlockSpec((1, nhid), lambda k: (0, 0)),
            pl.BlockSpec((nhid, nfeat), lambda k: (0, 0)),
            pl.BlockSpec((1, nfeat), lambda k: (0, 0)),
        ],
        out_specs=[
            pl.BlockSpec((npad, nfeat), lambda k: (0, 0)),
            pl.BlockSpec((rb, nfeat), lambda k: (k, 0)),
        ],
        out_shape=[
            jax.ShapeDtypeStruct((npad, nfeat), jnp.bfloat16),
            jax.ShapeDtypeStruct((n, nfeat), jnp.float32),
        ],
    )(adj, z1, b1.reshape(1, -1), W2, b2.reshape(1, -1))

    ks, js, fs = [], [], []
    for kk in range(ng):
        c_hi = ((kk + 1) * rb) // _CB
        for jj in range(min(c_hi, nchunk - 1), nchunk):
            ks.append(kk)
            js.append(jj)
            fs.append(1 if jj == min(c_hi, nchunk - 1) else 0)
    sarr = jnp.asarray(np.array([ks, js, fs], dtype=np.int32))
    nsteps = len(ks)

    grid_spec = pltpu.PrefetchScalarGridSpec(
        num_scalar_prefetch=1,
        grid=(nsteps,),
        in_specs=[
            pl.BlockSpec((rb, _CB), lambda t, s: (s[0, t], s[1, t])),
            pl.BlockSpec((rb, nfeat), lambda t, s: (s[0, t], 0)),
            pl.BlockSpec((npad, nfeat), lambda t, s: (0, 0)),
            pl.BlockSpec((nfeat, ncls), lambda t, s: (0, 0)),
            pl.BlockSpec((1, ncls), lambda t, s: (0, 0)),
        ],
        out_specs=[
            pl.BlockSpec((n, nfeat), lambda t, s: (0, 0)),
            pl.BlockSpec((rb, ncls), lambda t, s: (s[0, t], 0)),
        ],
        scratch_shapes=[
            pltpu.VMEM((n, nfeat), jnp.float32),
            pltpu.VMEM((1, nfeat), jnp.float32),
            pltpu.VMEM((1, nfeat), jnp.float32),
        ],
    )
    lsm, cls = pl.pallas_call(
        functools.partial(_pass_c_body, rb=rb, n=n, nsteps=nsteps,
                          nchunk=nchunk),
        grid_spec=grid_spec,
        out_shape=[
            jax.ShapeDtypeStruct((n, nfeat), jnp.float32),
            jax.ShapeDtypeStruct((n, ncls), jnp.float32),
        ],
    )(sarr, adj, part, z2, Wc, bc.reshape(1, -1))

    return (lsm, cls[:split], cls[split:])


# speedup vs baseline: 1.0759x; 1.0686x over previous
import functools

import jax
import jax.numpy as jnp
import numpy as np
from jax.experimental import pallas as pl
from jax.experimental.pallas import tpu as pltpu

_CB = 1280
_GRP = 5


def _mm(a, b):
    return jax.lax.dot_general(
        a.astype(jnp.bfloat16), b.astype(jnp.bfloat16),
        (((1,), (0,)), ((), ())),
        preferred_element_type=jnp.float32)


def _xw1_body(x_ref, w1_ref, z1_ref):
    z1_ref[...] = _mm(x_ref[...], w1_ref[...]).astype(jnp.bfloat16)


def _pass_b_body(adj_ref, z1_ref, b1_ref, w2_ref, b2_ref, z2_ref, part_ref,
                 *, rb, rc, n, nchunk):
    k = pl.program_id(0)

    @pl.when(k == 0)
    def _():
        z2_ref[...] = jnp.zeros_like(z2_ref)

    a = adj_ref[...].astype(jnp.bfloat16)
    pre = jax.lax.dot_general(
        a, z1_ref[...], (((1,), (0,)), ((), ())),
        preferred_element_type=jnp.float32) + b1_ref[...]
    h = jnp.maximum(pre, 0.0).astype(jnp.bfloat16)
    z2_ref[pl.ds(k * rb, rb), :] = _mm(h, w2_ref[...]).astype(jnp.bfloat16)

    cgrp = jnp.minimum(((k // (rc // rb)) * rc + rb) // _CB, nchunk - 1)
    part_ref[...] = jnp.zeros_like(part_ref) + b2_ref[...]
    for jc in range(nchunk - 1):
        @pl.when(jc < cgrp)
        def _(jc=jc):
            part_ref[...] = part_ref[...] + jax.lax.dot_general(
                a[:, jc * _CB:(jc + 1) * _CB],
                z2_ref[pl.ds(jc * _CB, _CB), :],
                (((1,), (0,)), ((), ())),
                preferred_element_type=jnp.float32)


def _pass_c_body(s_ref, adj_ref, part_ref, z2_ref, wc_ref, bc_ref,
                 lsm_ref, cls_ref, m_sc, sum_sc,
                 *, rc, n, nsteps, nchunk):
    t = pl.program_id(0)
    g = s_ref[0, t]
    jc = s_ref[1, t]
    first = s_ref[2, t] == 1
    tail = jc == nchunk - 1

    @pl.when(t == 0)
    def _():
        m_sc[...] = jnp.full_like(m_sc, -1e30)
        sum_sc[...] = jnp.zeros_like(sum_sc)

    a = adj_ref[...].astype(jnp.bfloat16)
    cols = jax.lax.broadcasted_iota(jnp.int32, (1, _CB), 1)
    a = jnp.where(cols < n - jc * _CB, a, jnp.bfloat16(0))
    contrib = jax.lax.dot_general(
        a, z2_ref[pl.ds(jc * _CB, _CB), :],
        (((1,), (0,)), ((), ())), preferred_element_type=jnp.float32)

    @pl.when(first)
    def _():
        lsm_ref[pl.ds(g * rc, rc), :] = part_ref[...] + contrib

    @pl.when(jnp.logical_not(first))
    def _():
        lsm_ref[pl.ds(g * rc, rc), :] = (
            lsm_ref[pl.ds(g * rc, rc), :] + contrib)

    @pl.when(tail)
    def _():
        acc = lsm_ref[pl.ds(g * rc, rc), :]
        cls_ref[...] = _mm(acc, wc_ref[...]) + bc_ref[...]
        m_old = m_sc[...]
        m_new = jnp.maximum(m_old, jnp.max(acc, axis=0, keepdims=True))
        sum_sc[...] = (sum_sc[...] * jnp.exp(m_old - m_new)
                       + jnp.sum(jnp.exp(acc - m_new), axis=0, keepdims=True))
        m_sc[...] = m_new

    @pl.when(t == nsteps - 1)
    def _():
        lse = m_sc[...] + jnp.log(sum_sc[...])
        lsm_ref[...] = lsm_ref[...] - lse


def kernel(x, adj, W1, b1, W2, b2, Wc, bc):
    n, nfeat = x.shape
    nhid = W1.shape[1]
    ncls = Wc.shape[1]
    split = 4576

    rb = 400 if n % 400 == 0 else n
    ng = n // rb
    grp = _GRP if ng % _GRP == 0 else 1
    rc = rb * grp
    ngrp = n // rc
    nchunk = -(-n // _CB)
    npad = nchunk * _CB

    z1 = pl.pallas_call(
        _xw1_body,
        out_shape=jax.ShapeDtypeStruct((n, nhid), jnp.bfloat16),
    )(x, W1)

    z2, part = pl.pallas_call(
        functools.partial(_pass_b_body, rb=rb, rc=rc, n=n, nchunk=nchunk),
        grid=(ng,),
        in_specs=[
            pl.BlockSpec((rb, n), lambda k: (k, 0)),
            pl.BlockSpec((n, nhid), lambda k: (0, 0)),
            pl.BlockSpec((1, nhid), lambda k: (0, 0)),
            pl.BlockSpec((nhid, nfeat), lambda k: (0, 0)),
            pl.BlockSpec((1, nfeat), lambda k: (0, 0)),
        ],
        out_specs=[
            pl.BlockSpec((npad, nfeat), lambda k: (0, 0)),
            pl.BlockSpec((rb, nfeat), lambda k: (k, 0)),
        ],
        out_shape=[
            jax.ShapeDtypeStruct((npad, nfeat), jnp.bfloat16),
            jax.ShapeDtypeStruct((n, nfeat), jnp.float32),
        ],
    )(adj, z1, b1.reshape(1, -1), W2, b2.reshape(1, -1))

    gs, js, fs = [], [], []
    for gg in range(ngrp):
        cgrp = min((gg * rc + rb) // _CB, nchunk - 1)
        for jj in range(cgrp, nchunk):
            gs.append(gg)
            js.append(jj)
            fs.append(1 if jj == cgrp else 0)
    sarr = jnp.asarray(np.array([gs, js, fs], dtype=np.int32))
    nsteps = len(gs)

    grid_spec = pltpu.PrefetchScalarGridSpec(
        num_scalar_prefetch=1,
        grid=(nsteps,),
        in_specs=[
            pl.BlockSpec((rc, _CB), lambda t, s: (s[0, t], s[1, t])),
            pl.BlockSpec((rc, nfeat), lambda t, s: (s[0, t], 0)),
            pl.BlockSpec((npad, nfeat), lambda t, s: (0, 0)),
            pl.BlockSpec((nfeat, ncls), lambda t, s: (0, 0)),
            pl.BlockSpec((1, ncls), lambda t, s: (0, 0)),
        ],
        out_specs=[
            pl.BlockSpec((n, nfeat), lambda t, s: (0, 0)),
            pl.BlockSpec((rc, ncls), lambda t, s: (s[0, t], 0)),
        ],
        scratch_shapes=[
            pltpu.VMEM((1, nfeat), jnp.float32),
            pltpu.VMEM((1, nfeat), jnp.float32),
        ],
    )
    lsm, cls = pl.pallas_call(
        functools.partial(_pass_c_body, rc=rc, n=n, nsteps=nsteps,
                          nchunk=nchunk),
        grid_spec=grid_spec,
        out_shape=[
            jax.ShapeDtypeStruct((n, nfeat), jnp.float32),
            jax.ShapeDtypeStruct((n, ncls), jnp.float32),
        ],
    )(sarr, adj, part, z2, Wc, bc.reshape(1, -1))

    return (lsm, cls[:split], cls[split:])


# speedup vs baseline: 1.0812x; 1.0049x over previous
import functools

import jax
import jax.numpy as jnp
import numpy as np
from jax.experimental import pallas as pl
from jax.experimental.pallas import tpu as pltpu

_CB = 1280
_GRP = 5


def _mm(a, b):
    return jax.lax.dot_general(
        a.astype(jnp.bfloat16), b.astype(jnp.bfloat16),
        (((1,), (0,)), ((), ())),
        preferred_element_type=jnp.float32)


def _xw1_body(x_ref, w1_ref, z1_ref):
    z1_ref[...] = _mm(x_ref[...], w1_ref[...]).astype(jnp.bfloat16)


def _pass_b_body(adj_ref, z1_ref, b1_ref, w2_ref, b2_ref, z2_ref, part_ref,
                 acc_sc, *, rb, rc, n, nchunk):
    k = pl.program_id(0)

    @pl.when(k == 0)
    def _():
        z2_ref[...] = jnp.zeros_like(z2_ref)

    a = adj_ref[...].astype(jnp.bfloat16)
    pre = jax.lax.dot_general(
        a, z1_ref[...], (((1,), (0,)), ((), ())),
        preferred_element_type=jnp.float32) + b1_ref[...]
    h = jnp.maximum(pre, 0.0).astype(jnp.bfloat16)
    z2_ref[pl.ds(k * rb, rb), :] = _mm(h, w2_ref[...]).astype(jnp.bfloat16)

    cgrp = jnp.minimum(((k // (rc // rb)) * rc + rb) // _CB, nchunk - 1)
    acc_sc[...] = jnp.zeros_like(acc_sc) + b2_ref[...]
    for jc in range(nchunk - 1):
        @pl.when(jc < cgrp)
        def _(jc=jc):
            acc_sc[...] = acc_sc[...] + jax.lax.dot_general(
                a[:, jc * _CB:(jc + 1) * _CB],
                z2_ref[pl.ds(jc * _CB, _CB), :],
                (((1,), (0,)), ((), ())),
                preferred_element_type=jnp.float32)
    part_ref[...] = acc_sc[...].astype(jnp.bfloat16)


def _pass_c_body(s_ref, adj_ref, part_ref, z2_ref, wc_ref, bc_ref,
                 lsm_ref, cls_ref, m_sc, sum_sc,
                 *, rc, n, nsteps, nchunk):
    t = pl.program_id(0)
    g = s_ref[0, t]
    jc = s_ref[1, t]
    first = s_ref[2, t] == 1
    tail = jc == nchunk - 1

    @pl.when(t == 0)
    def _():
        m_sc[...] = jnp.full_like(m_sc, -1e30)
        sum_sc[...] = jnp.zeros_like(sum_sc)

    a = adj_ref[...].astype(jnp.bfloat16)
    cols = jax.lax.broadcasted_iota(jnp.int32, (1, _CB), 1)
    a = jnp.where(cols < n - jc * _CB, a, jnp.bfloat16(0))
    contrib = jax.lax.dot_general(
        a, z2_ref[pl.ds(jc * _CB, _CB), :],
        (((1,), (0,)), ((), ())), preferred_element_type=jnp.float32)

    @pl.when(first)
    def _():
        lsm_ref[pl.ds(g * rc, rc), :] = part_ref[...] + contrib

    @pl.when(jnp.logical_not(first))
    def _():
        lsm_ref[pl.ds(g * rc, rc), :] = (
            lsm_ref[pl.ds(g * rc, rc), :] + contrib)

    @pl.when(tail)
    def _():
        acc = lsm_ref[pl.ds(g * rc, rc), :]
        cls_ref[...] = _mm(acc, wc_ref[...]) + bc_ref[...]
        m_old = m_sc[...]
        m_new = jnp.maximum(m_old, jnp.max(acc, axis=0, keepdims=True))
        sum_sc[...] = (sum_sc[...] * jnp.exp(m_old - m_new)
                       + jnp.sum(jnp.exp(acc - m_new), axis=0, keepdims=True))
        m_sc[...] = m_new

    @pl.when(t == nsteps - 1)
    def _():
        lse = m_sc[...] + jnp.log(sum_sc[...])
        lsm_ref[...] = lsm_ref[...] - lse


def kernel(x, adj, W1, b1, W2, b2, Wc, bc):
    n, nfeat = x.shape
    nhid = W1.shape[1]
    ncls = Wc.shape[1]
    split = 4576

    rb = 400 if n % 400 == 0 else n
    ng = n // rb
    grp = _GRP if ng % _GRP == 0 else 1
    rc = rb * grp
    ngrp = n // rc
    nchunk = -(-n // _CB)
    npad = nchunk * _CB

    z1 = pl.pallas_call(
        _xw1_body,
        out_shape=jax.ShapeDtypeStruct((n, nhid), jnp.bfloat16),
    )(x, W1)

    z2, part = pl.pallas_call(
        functools.partial(_pass_b_body, rb=rb, rc=rc, n=n, nchunk=nchunk),
        grid=(ng,),
        in_specs=[
            pl.BlockSpec((rb, n), lambda k: (k, 0)),
            pl.BlockSpec((n, nhid), lambda k: (0, 0)),
            pl.BlockSpec((1, nhid), lambda k: (0, 0)),
            pl.BlockSpec((nhid, nfeat), lambda k: (0, 0)),
            pl.BlockSpec((1, nfeat), lambda k: (0, 0)),
        ],
        out_specs=[
            pl.BlockSpec((npad, nfeat), lambda k: (0, 0)),
            pl.BlockSpec((rb, nfeat), lambda k: (k, 0)),
        ],
        out_shape=[
            jax.ShapeDtypeStruct((npad, nfeat), jnp.bfloat16),
            jax.ShapeDtypeStruct((n, nfeat), jnp.bfloat16),
        ],
        scratch_shapes=[pltpu.VMEM((rb, nfeat), jnp.float32)],
    )(adj, z1, b1.reshape(1, -1), W2, b2.reshape(1, -1))

    gs, js, fs = [], [], []
    for gg in range(ngrp):
        cgrp = min((gg * rc + rb) // _CB, nchunk - 1)
        for jj in range(cgrp, nchunk):
            gs.append(gg)
            js.append(jj)
            fs.append(1 if jj == cgrp else 0)
    sarr = jnp.asarray(np.array([gs, js, fs], dtype=np.int32))
    nsteps = len(gs)

    grid_spec = pltpu.PrefetchScalarGridSpec(
        num_scalar_prefetch=1,
        grid=(nsteps,),
        in_specs=[
            pl.BlockSpec((rc, _CB), lambda t, s: (s[0, t], s[1, t])),
            pl.BlockSpec((rc, nfeat), lambda t, s: (s[0, t], 0)),
            pl.BlockSpec((npad, nfeat), lambda t, s: (0, 0)),
            pl.BlockSpec((nfeat, ncls), lambda t, s: (0, 0)),
            pl.BlockSpec((1, ncls), lambda t, s: (0, 0)),
        ],
        out_specs=[
            pl.BlockSpec((n, nfeat), lambda t, s: (0, 0)),
            pl.BlockSpec((rc, ncls), lambda t, s: (s[0, t], 0)),
        ],
        scratch_shapes=[
            pltpu.VMEM((1, nfeat), jnp.float32),
            pltpu.VMEM((1, nfeat), jnp.float32),
        ],
    )
    lsm, cls = pl.pallas_call(
        functools.partial(_pass_c_body, rc=rc, n=n, nsteps=nsteps,
                          nchunk=nchunk),
        grid_spec=grid_spec,
        out_shape=[
            jax.ShapeDtypeStruct((n, nfeat), jnp.float32),
            jax.ShapeDtypeStruct((n, ncls), jnp.float32),
        ],
    )(sarr, adj, part, z2, Wc, bc.reshape(1, -1))

    return (lsm, cls[:split], cls[split:])
